# baseline (speedup 1.0000x reference)
import functools
import math

import numpy as np
import jax
import jax.numpy as jnp
from jax import lax
from jax.experimental import pallas as pl
from jax.experimental.pallas import tpu as pltpu
from jax.experimental.pallas import tpu_sc as plsc

L_SEQ = 50
BATCH = 4096
NE = 460
NE_PAD = 464
POS = 26
OUT_D = 512
N_AA = 27
N_MOD = 8
N_KEY = N_AA * N_MOD
N_PAIR = 51 * 51
PAIR_W = 128
ROWS = L_SEQ * BATCH
SQRT_NE = math.sqrt(float(NE))
REP = 16


def _make_pe() -> np.ndarray:
    max_len, dims = 128, POS
    position = np.arange(0, max_len, dtype=np.float32)[:, None]
    div_term_enum = np.arange(0, dims, 2, dtype=np.float32)
    div_term_denom = -math.log(10000.0) / dims + 1
    div_term = np.exp(div_term_enum * div_term_denom)
    pe = np.zeros((max_len, dims), dtype=np.float32)
    pe[:, 0::2] = np.sin(position * div_term)
    pe[:, 1::2] = np.cos(position * div_term)
    pe[0, :] = 0.0
    return pe


def _make_pairs() -> np.ndarray:
    pe51 = _make_pe()[:51]
    pairs = np.zeros((51, 51, PAIR_W), dtype=np.float32)
    pairs[:, :, 76:76 + POS] = pe51[:, None, :]
    pairs[:, :, 76 + POS:] = pe51[None, :, :]
    return np.tile(pairs.reshape(N_PAIR, PAIR_W), (REP, 1))


_PAIRS = _make_pairs()


def _prep_body(src_ref, mods_ref, aa_ref, modt_ref, key_ref, sum_ref):
    srcf = src_ref[...].astype(jnp.float32)
    modsf = mods_ref[...].astype(jnp.float32)
    ks_t = jnp.transpose(srcf * 8.0 + modsf).astype(jnp.int32)

    mask_t = jnp.transpose(jnp.where(srcf != 0.0, 1.0, 0.0))
    r_io = lax.broadcasted_iota(jnp.int32, (L_SEQ, L_SEQ), 0)
    c_io = lax.broadcasted_iota(jnp.int32, (L_SEQ, L_SEQ), 1)
    lower = jnp.where(r_io >= c_io, 1.0, 0.0)
    fw_t = jnp.dot(lower, mask_t, preferred_element_type=jnp.float32)
    tot_t = fw_t[L_SEQ - 1:L_SEQ, :]
    rev_t = tot_t - fw_t + mask_t
    kp_t = (fw_t * 51.0 + rev_t).astype(jnp.int32)

    l_io = lax.broadcasted_iota(jnp.int32, (L_SEQ, BATCH), 0)
    b_io = lax.broadcasted_iota(jnp.int32, (L_SEQ, BATCH), 1)
    row_io = l_io * BATCH + b_io
    rep_io = lax.bitwise_and(lax.shift_right_logical(row_io, 8), REP - 1)
    kp_t = kp_t + rep_io * N_PAIR
    key_ref[...] = lax.shift_left(ks_t, 16) + kp_t

    fused = aa_ref[...][:, None, :] + modt_ref[...][None, :, :]
    sum_ref[...] = fused.reshape(N_KEY, NE) * SQRT_NE


def _prep(src, mods, aa_table, mod_table):
    return pl.pallas_call(
        _prep_body,
        out_shape=(
            jax.ShapeDtypeStruct((L_SEQ, BATCH), jnp.int32),
            jax.ShapeDtypeStruct((N_KEY, NE), jnp.float32),
        ),
    )(src, mods, aa_table, mod_table)


def _sc_info():
    try:
        info = plsc.get_sparse_core_info()
        return info.num_cores, info.num_subcores
    except Exception:
        return 2, 16


NC, NS = _sc_info()
NW = NC * NS
RPW = ROWS // NW
CHUNK = 16
NCHUNK = RPW // CHUNK
NPAIRS = NCHUNK // 2


def _sc_body(sum_hbm, pairs_hbm, key_hbm, out_hbm,
             sum_v, key_v, a_vs, p_vs, kp_vs, sb_s, sw_s):
    wid = lax.axis_index("s") * NC + lax.axis_index("c")
    base = wid * RPW
    pltpu.sync_copy(sum_hbm, sum_v.at[pl.ds(0, N_KEY * NE)])
    pltpu.sync_copy(key_hbm.at[pl.ds(base, RPW)], key_v)

    lane = lax.iota(jnp.int32, 16)
    tail_keep = lane < 12

    def do_chunk(g, s):
        off = g * CHUNK
        kv = key_v[pl.ds(off, CHUNK)]
        kp_vs[s][...] = lax.bitwise_and(kv, 0xFFFF)
        pb = pltpu.async_copy(pairs_hbm.at[kp_vs[s]], p_vs[s], sb_s[s])

        @pl.when(g >= 2)
        def _():
            pltpu.make_async_copy(
                a_vs[s], out_hbm.at[pl.ds(base + off, CHUNK)], sw_s[s]).wait()

        bases = [lax.shift_right_logical(kv[r], 16) * NE for r in range(CHUNK)]
        for r in range(CHUNK):
            for j in range(28):
                a_vs[s][r, pl.ds(16 * j, 16)] = sum_v[pl.ds(bases[r] + 16 * j, 16)]
        pb.wait()
        for r in range(CHUNK):
            tail = jnp.where(tail_keep, sum_v[pl.ds(bases[r] + 448, 16)], 0.0)
            a_vs[s][r, pl.ds(448, 16)] = tail + p_vs[s][r, pl.ds(64, 16)]
            for j in (1, 2, 3):
                a_vs[s][r, pl.ds(448 + 16 * j, 16)] = p_vs[s][r, pl.ds(64 + 16 * j, 16)]
        pltpu.async_copy(
            a_vs[s], out_hbm.at[pl.ds(base + off, CHUNK)], sw_s[s])

    def pair(i, carry):
        do_chunk(2 * i, 0)
        do_chunk(2 * i + 1, 1)
        return carry

    lax.fori_loop(0, NPAIRS, pair, 0)
    for s in range(2):
        g = NCHUNK - 2 + s
        pltpu.make_async_copy(
            a_vs[s], out_hbm.at[pl.ds(base + g * CHUNK, CHUNK)], sw_s[s]).wait()


@functools.partial(jax.jit, static_argnums=())
def _run_sc(sum464, pairs, keys):
    mesh = plsc.VectorSubcoreMesh(core_axis_name="c", subcore_axis_name="s",
                                  num_cores=NC, num_subcores=NS)

    def body(sum_hbm, pairs_hbm, key_hbm, out_hbm, *scratch):
        sum_v, key_v = scratch[0], scratch[1]
        a_vs = scratch[2:4]
        p_vs = scratch[4:6]
        kp_vs = scratch[6:8]
        sb_s = scratch[8:10]
        sw_s = scratch[10:12]
        _sc_body(sum_hbm, pairs_hbm, key_hbm, out_hbm,
                 sum_v, key_v, a_vs, p_vs, kp_vs, sb_s, sw_s)

    f = pl.kernel(
        body,
        out_type=jax.ShapeDtypeStruct((ROWS, OUT_D), jnp.float32),
        mesh=mesh,
        scratch_types=(
            [pltpu.VMEM((N_KEY * NE + 16,), jnp.float32),
             pltpu.VMEM((RPW,), jnp.int32)]
            + [pltpu.VMEM((CHUNK, OUT_D), jnp.float32) for _ in range(2)]
            + [pltpu.VMEM((CHUNK, PAIR_W), jnp.float32) for _ in range(2)]
            + [pltpu.VMEM((CHUNK,), jnp.int32) for _ in range(2)]
            + [pltpu.SemaphoreType.DMA for _ in range(4)]
        ),
    )
    return f(sum464, pairs, keys)


def kernel(src, mods, aa_table, mod_table):
    src = src.astype(jnp.int32)
    mods = mods.astype(jnp.int32)
    keys, sum464 = _prep(src, mods, aa_table, mod_table)
    pairs = jnp.asarray(_PAIRS)
    out = _run_sc(sum464.reshape(N_KEY * NE), pairs, keys.reshape(ROWS))
    return out.reshape(L_SEQ, BATCH, OUT_D)


# speedup vs baseline: 1.2615x; 1.2615x over previous
import functools
import math

import numpy as np
import jax
import jax.numpy as jnp
from jax import lax
from jax.experimental import pallas as pl
from jax.experimental.pallas import tpu as pltpu
from jax.experimental.pallas import tpu_sc as plsc

L_SEQ = 50
BATCH = 4096
NE = 460
NE_PAD = 464
POS = 26
OUT_D = 512
N_AA = 27
N_MOD = 8
N_KEY = N_AA * N_MOD
N_PAIR = 51 * 51
PAIR_W = 128
ROWS = L_SEQ * BATCH
SQRT_NE = math.sqrt(float(NE))
REP = 16


def _make_pe() -> np.ndarray:
    max_len, dims = 128, POS
    position = np.arange(0, max_len, dtype=np.float32)[:, None]
    div_term_enum = np.arange(0, dims, 2, dtype=np.float32)
    div_term_denom = -math.log(10000.0) / dims + 1
    div_term = np.exp(div_term_enum * div_term_denom)
    pe = np.zeros((max_len, dims), dtype=np.float32)
    pe[:, 0::2] = np.sin(position * div_term)
    pe[:, 1::2] = np.cos(position * div_term)
    pe[0, :] = 0.0
    return pe


def _make_pairs() -> np.ndarray:
    pe51 = _make_pe()[:51]
    pairs = np.zeros((51, 51, PAIR_W), dtype=np.float32)
    pairs[:, :, 76:76 + POS] = pe51[:, None, :]
    pairs[:, :, 76 + POS:] = pe51[None, :, :]
    return np.tile(pairs.reshape(N_PAIR, PAIR_W), (REP, 1))


_PAIRS = _make_pairs()


def _prep_body(src_ref, mods_ref, aa_ref, modt_ref, key_ref, sum_ref):
    srcf = src_ref[...].astype(jnp.float32)
    modsf = mods_ref[...].astype(jnp.float32)
    ks_t = jnp.transpose(srcf * 8.0 + modsf).astype(jnp.int32)

    mask_t = jnp.transpose(jnp.where(srcf != 0.0, 1.0, 0.0))
    r_io = lax.broadcasted_iota(jnp.int32, (L_SEQ, L_SEQ), 0)
    c_io = lax.broadcasted_iota(jnp.int32, (L_SEQ, L_SEQ), 1)
    lower = jnp.where(r_io >= c_io, 1.0, 0.0)
    fw_t = jnp.dot(lower, mask_t, preferred_element_type=jnp.float32)
    tot_t = fw_t[L_SEQ - 1:L_SEQ, :]
    rev_t = tot_t - fw_t + mask_t
    kp_t = (fw_t * 51.0 + rev_t).astype(jnp.int32)

    l_io = lax.broadcasted_iota(jnp.int32, (L_SEQ, BATCH), 0)
    b_io = lax.broadcasted_iota(jnp.int32, (L_SEQ, BATCH), 1)
    row_io = l_io * BATCH + b_io
    rep_io = lax.bitwise_and(lax.shift_right_logical(row_io, 8), REP - 1)
    kp_t = kp_t + rep_io * N_PAIR
    key_ref[...] = lax.shift_left(ks_t, 16) + kp_t

    fused = aa_ref[...][:, None, :] + modt_ref[...][None, :, :]
    sum_ref[...] = fused.reshape(N_KEY, NE) * SQRT_NE


def _prep(src, mods, aa_table, mod_table):
    return pl.pallas_call(
        _prep_body,
        out_shape=(
            jax.ShapeDtypeStruct((L_SEQ, BATCH), jnp.int32),
            jax.ShapeDtypeStruct((N_KEY, NE), jnp.float32),
        ),
    )(src, mods, aa_table, mod_table)


def _sc_info():
    try:
        info = plsc.get_sparse_core_info()
        return info.num_cores, info.num_subcores
    except Exception:
        return 2, 16


NC, NS = _sc_info()
NW = NC * NS
RPW = ROWS // NW
CHUNK = 16
NCHUNK = RPW // CHUNK
NPAIRS = NCHUNK // 2


def _sc_body(sum_hbm, pairs_hbm, key_hbm, out_hbm,
             sum_v, key_v, a_vs, p_vs, kp_vs, sb_s, sw_s):
    wid = lax.axis_index("s") * NC + lax.axis_index("c")
    base = wid * RPW
    pltpu.sync_copy(sum_hbm, sum_v.at[pl.ds(0, N_KEY * NE)])
    pltpu.sync_copy(key_hbm.at[pl.ds(base, RPW)], key_v)

    lane = lax.iota(jnp.int32, 16)
    tail_keep = lane < 12

    def do_chunk(g, s):
        off = g * CHUNK
        kv = key_v[pl.ds(off, CHUNK)]
        kp_vs[s][...] = lax.bitwise_and(kv, 0xFFFF)
        pb = pltpu.async_copy(pairs_hbm.at[kp_vs[s]], p_vs[s], sb_s[s])

        @pl.when(g >= 2)
        def _():
            pltpu.make_async_copy(
                a_vs[s], out_hbm.at[pl.ds(base + off, CHUNK)], sw_s[s]).wait()

        bases = [lax.shift_right_logical(kv[r], 16) * NE for r in range(CHUNK)]
        for r in range(CHUNK):
            vals = [sum_v[pl.ds(bases[r] + 16 * j, 16)] for j in range(28)]
            for j in range(28):
                a_vs[s][r, pl.ds(16 * j, 16)] = vals[j]
        pb.wait()
        for r in range(CHUNK):
            tail = jnp.where(tail_keep, sum_v[pl.ds(bases[r] + 448, 16)], 0.0)
            a_vs[s][r, pl.ds(448, 16)] = tail + p_vs[s][r, pl.ds(64, 16)]
            for j in (1, 2, 3):
                a_vs[s][r, pl.ds(448 + 16 * j, 16)] = p_vs[s][r, pl.ds(64 + 16 * j, 16)]
        pltpu.async_copy(
            a_vs[s], out_hbm.at[pl.ds(base + off, CHUNK)], sw_s[s])

    def pair(i, carry):
        do_chunk(2 * i, 0)
        do_chunk(2 * i + 1, 1)
        return carry

    lax.fori_loop(0, NPAIRS, pair, 0)
    for s in range(2):
        g = NCHUNK - 2 + s
        pltpu.make_async_copy(
            a_vs[s], out_hbm.at[pl.ds(base + g * CHUNK, CHUNK)], sw_s[s]).wait()


@functools.partial(jax.jit, static_argnums=())
def _run_sc(sum464, pairs, keys):
    mesh = plsc.VectorSubcoreMesh(core_axis_name="c", subcore_axis_name="s",
                                  num_cores=NC, num_subcores=NS)

    def body(sum_hbm, pairs_hbm, key_hbm, out_hbm, *scratch):
        sum_v, key_v = scratch[0], scratch[1]
        a_vs = scratch[2:4]
        p_vs = scratch[4:6]
        kp_vs = scratch[6:8]
        sb_s = scratch[8:10]
        sw_s = scratch[10:12]
        _sc_body(sum_hbm, pairs_hbm, key_hbm, out_hbm,
                 sum_v, key_v, a_vs, p_vs, kp_vs, sb_s, sw_s)

    f = pl.kernel(
        body,
        out_type=jax.ShapeDtypeStruct((ROWS, OUT_D), jnp.float32),
        mesh=mesh,
        scratch_types=(
            [pltpu.VMEM((N_KEY * NE + 16,), jnp.float32),
             pltpu.VMEM((RPW,), jnp.int32)]
            + [pltpu.VMEM((CHUNK, OUT_D), jnp.float32) for _ in range(2)]
            + [pltpu.VMEM((CHUNK, PAIR_W), jnp.float32) for _ in range(2)]
            + [pltpu.VMEM((CHUNK,), jnp.int32) for _ in range(2)]
            + [pltpu.SemaphoreType.DMA for _ in range(4)]
        ),
    )
    return f(sum464, pairs, keys)


def kernel(src, mods, aa_table, mod_table):
    src = src.astype(jnp.int32)
    mods = mods.astype(jnp.int32)
    keys, sum464 = _prep(src, mods, aa_table, mod_table)
    pairs = jnp.asarray(_PAIRS)
    out = _run_sc(sum464.reshape(N_KEY * NE), pairs, keys.reshape(ROWS))
    return out.reshape(L_SEQ, BATCH, OUT_D)
